# Initial kernel scaffold; baseline (speedup 1.0000x reference)
import jax
import jax.numpy as jnp
from jax.experimental import pallas as pl


def kernel(x, node_types, edge_index0, edge_index1, Win, b_in, Wc0, bc0, Wc10, bc10, Wc11, bc11, W1, b1, W2, b2, W3, b3):
    raise NotImplementedError("write your pallas kernel here")


# speedup vs baseline: 16.4189x; 16.4189x over previous
import functools
import math

import jax
import jax.numpy as jnp
from jax import lax
from jax.experimental import pallas as pl
from jax.experimental.pallas import tpu as pltpu
from jax.experimental.pallas import tpu_sc as plsc

N = 10000
E = 320000
H = 128
C = 16

NC = 2
NS = 16
NW = NC * NS
EPW = E // NW
EB = 80
NBATCH = EPW // EB
NP = 10240
RPS = NP // NS
HW = 16

_mesh = plsc.VectorSubcoreMesh(core_axis_name="c", subcore_axis_name="s")


@functools.partial(
    pl.kernel,
    out_type=(
        jax.ShapeDtypeStruct((NC, NP, HW), jnp.float32),
        jax.ShapeDtypeStruct((NC, NP, HW), jnp.float32),
    ),
    mesh=_mesh,
    compiler_params=pltpu.CompilerParams(use_tc_tiling_on_sc=False),
    scratch_types=[
        pltpu.VMEM((EB,), jnp.int32),
        pltpu.VMEM((EB, HW), jnp.float32),
        pltpu.VMEM((RPS, HW), jnp.float32),
        pltpu.VMEM_SHARED((NP, HW), jnp.float32),
        pltpu.VMEM_SHARED((NP, HW), jnp.float32),
    ],
)
def _sc_hist(dst0_hbm, dst1_hbm, out0_hbm, out1_hbm,
             idx_v, ones_v, zb_v, t0_sh, t1_sh):
    c = lax.axis_index("c")
    s = lax.axis_index("s")
    wid = c * NS + s

    def fill(i, _):
        ones_v[i, :] = jnp.ones((HW,), jnp.float32)
        return _
    lax.fori_loop(0, EB, fill, None)

    def zero(i, _):
        zb_v[i, :] = jnp.zeros((HW,), jnp.float32)
        return _
    lax.fori_loop(0, RPS, zero, None)

    row0 = s * RPS
    pltpu.sync_copy(zb_v, t0_sh.at[pl.ds(row0, RPS), :])
    pltpu.sync_copy(zb_v, t1_sh.at[pl.ds(row0, RPS), :])
    plsc.subcore_barrier()

    base = wid * EPW

    def batch(j, _):
        off = pl.multiple_of(base + j * EB, 8)
        pltpu.sync_copy(dst0_hbm.at[pl.ds(off, EB)], idx_v)
        pltpu.sync_copy(ones_v, t0_sh.at[idx_v], add=True)
        pltpu.sync_copy(dst1_hbm.at[pl.ds(off, EB)], idx_v)
        pltpu.sync_copy(ones_v, t1_sh.at[idx_v], add=True)
        return _
    lax.fori_loop(0, NBATCH, batch, None)

    plsc.subcore_barrier()
    pltpu.sync_copy(t0_sh.at[pl.ds(row0, RPS), :],
                    out0_hbm.at[c, pl.ds(row0, RPS), :])
    pltpu.sync_copy(t1_sh.at[pl.ds(row0, RPS), :],
                    out1_hbm.at[c, pl.ds(row0, RPS), :])


@functools.partial(
    pl.kernel,
    out_type=jax.ShapeDtypeStruct((NC, NP, H), jnp.float32),
    mesh=_mesh,
    compiler_params=pltpu.CompilerParams(use_tc_tiling_on_sc=False),
    scratch_types=[
        pltpu.VMEM((EB,), jnp.int32),
        pltpu.VMEM((EB,), jnp.int32),
        pltpu.VMEM((EB, H), jnp.float32),
        pltpu.VMEM((RPS // 5, H), jnp.float32),
        pltpu.VMEM_SHARED((NP, H), jnp.float32),
        pltpu.SemaphoreType.DMA,
    ],
)
def _sc_seg(y_hbm, src_hbm, dst_hbm, out_hbm,
            src_v, dst_v, rows_v, zb_v, table_sh, sem):
    c = lax.axis_index("c")
    s = lax.axis_index("s")
    wid = c * NS + s

    zrows = RPS // 5

    def zero(i, _):
        for k in range(H // 16):
            zb_v[i, pl.ds(k * 16, 16)] = jnp.zeros((16,), jnp.float32)
        return _
    lax.fori_loop(0, zrows, zero, None)

    row0 = s * RPS
    for r in range(5):
        pltpu.sync_copy(zb_v, table_sh.at[pl.ds(row0 + r * zrows, zrows), :])
    plsc.subcore_barrier()

    base = wid * EPW

    def batch(j, _):
        off = pl.multiple_of(base + j * EB, 8)
        pltpu.sync_copy(src_hbm.at[pl.ds(off, EB)], src_v)
        pltpu.sync_copy(dst_hbm.at[pl.ds(off, EB)], dst_v)
        pltpu.async_copy(y_hbm.at[src_v], rows_v, sem).wait()
        pltpu.sync_copy(rows_v, table_sh.at[dst_v], add=True)
        return _
    lax.fori_loop(0, NBATCH, batch, None)

    plsc.subcore_barrier()
    pltpu.sync_copy(table_sh.at[pl.ds(row0, RPS), :],
                    out_hbm.at[c, pl.ds(row0, RPS), :])


BLK = 1000
GRID = N // BLK


def _tc1_body(x_ref, nt_ref, h0_ref, h1_ref, win_ref, bin_ref,
              y0_ref, d0_ref, d1_ref):
    x = x_ref[...]
    nt = nt_ref[...]
    hid = jnp.zeros_like(x)
    for t in range(4):
        p = lax.dot_general(x, win_ref[t], (((1,), (1,)), ((), ())),
                            preferred_element_type=jnp.float32)
        p = p + bin_ref[t][None, :]
        hid = jnp.where(nt == t, p, hid)
    deg0 = h0_ref[0, :, 0:1] + h0_ref[1, :, 0:1] + 1.0
    deg1 = h1_ref[0, :, 0:1] + h1_ref[1, :, 0:1] + 1.0
    d0 = lax.rsqrt(deg0)
    d1 = lax.rsqrt(deg1)
    y0_ref[...] = hid * d0
    d0_ref[...] = d0
    d1_ref[...] = d1


def _tc1(x, nt2d, h0t, h1t, Win, b_in):
    return pl.pallas_call(
        _tc1_body,
        grid=(GRID,),
        in_specs=[
            pl.BlockSpec((BLK, H), lambda i: (i, 0)),
            pl.BlockSpec((BLK, 1), lambda i: (i, 0)),
            pl.BlockSpec((NC, BLK, HW), lambda i: (0, i, 0)),
            pl.BlockSpec((NC, BLK, HW), lambda i: (0, i, 0)),
            pl.BlockSpec((4, H, H), lambda i: (0, 0, 0)),
            pl.BlockSpec((4, H), lambda i: (0, 0)),
        ],
        out_specs=[
            pl.BlockSpec((BLK, H), lambda i: (i, 0)),
            pl.BlockSpec((BLK, 1), lambda i: (i, 0)),
            pl.BlockSpec((BLK, 1), lambda i: (i, 0)),
        ],
        out_shape=[
            jax.ShapeDtypeStruct((N, H), jnp.float32),
            jax.ShapeDtypeStruct((N, 1), jnp.float32),
            jax.ShapeDtypeStruct((N, 1), jnp.float32),
        ],
    )(x, nt2d, h0t, h1t, Win, b_in)


def _tc2_body(y0_ref, s0_ref, d0_ref, d1_ref, wc0_ref, bc0_ref,
              wc10_ref, bc10_ref, part1_ref, y1_ref):
    agg0 = d0_ref[...] * (s0_ref[0] + s0_ref[1] + y0_ref[...])
    h0 = lax.dot_general(agg0, wc0_ref[...], (((1,), (1,)), ((), ())),
                         preferred_element_type=jnp.float32) + bc0_ref[...]
    part1 = lax.dot_general(agg0, wc10_ref[...], (((1,), (1,)), ((), ())),
                            preferred_element_type=jnp.float32) + bc10_ref[...]
    part1_ref[...] = part1
    y1_ref[...] = d1_ref[...] * h0


def _tc2(y0, s0, d0, d1, Wc0, bc0, Wc10, bc10):
    return pl.pallas_call(
        _tc2_body,
        grid=(GRID,),
        in_specs=[
            pl.BlockSpec((BLK, H), lambda i: (i, 0)),
            pl.BlockSpec((NC, BLK, H), lambda i: (0, i, 0)),
            pl.BlockSpec((BLK, 1), lambda i: (i, 0)),
            pl.BlockSpec((BLK, 1), lambda i: (i, 0)),
            pl.BlockSpec((H, H), lambda i: (0, 0)),
            pl.BlockSpec((1, H), lambda i: (0, 0)),
            pl.BlockSpec((H, H), lambda i: (0, 0)),
            pl.BlockSpec((1, H), lambda i: (0, 0)),
        ],
        out_specs=[
            pl.BlockSpec((BLK, H), lambda i: (i, 0)),
            pl.BlockSpec((BLK, H), lambda i: (i, 0)),
        ],
        out_shape=[
            jax.ShapeDtypeStruct((N, H), jnp.float32),
            jax.ShapeDtypeStruct((N, H), jnp.float32),
        ],
    )(y0, s0, d0, d1, Wc0, bc0, Wc10, bc10)


def _tc3_body(part1_ref, y1_ref, s1_ref, d1_ref, wc11_ref, bc11_ref,
              w3_ref, b3_ref, out_ref):
    agg1 = d1_ref[...] * (s1_ref[0] + s1_ref[1] + y1_ref[...])
    h1 = part1_ref[...] + lax.dot_general(
        agg1, wc11_ref[...], (((1,), (1,)), ((), ())),
        preferred_element_type=jnp.float32) + bc11_ref[...]
    mu = jnp.mean(h1, axis=-1, keepdims=True)
    cent = h1 - mu
    var = jnp.mean(cent * cent, axis=-1, keepdims=True)
    tnorm = cent * lax.rsqrt(var + 1e-5)
    g = 0.5 * tnorm * (1.0 + lax.erf(tnorm * (1.0 / math.sqrt(2.0))))
    out_ref[...] = lax.dot_general(
        g, w3_ref[...], (((1,), (1,)), ((), ())),
        preferred_element_type=jnp.float32) + b3_ref[...]


def _tc3(part1, y1, s1, d1, Wc11, bc11, W3, b3):
    return pl.pallas_call(
        _tc3_body,
        grid=(GRID,),
        in_specs=[
            pl.BlockSpec((BLK, H), lambda i: (i, 0)),
            pl.BlockSpec((BLK, H), lambda i: (i, 0)),
            pl.BlockSpec((NC, BLK, H), lambda i: (0, i, 0)),
            pl.BlockSpec((BLK, 1), lambda i: (i, 0)),
            pl.BlockSpec((H, H), lambda i: (0, 0)),
            pl.BlockSpec((1, H), lambda i: (0, 0)),
            pl.BlockSpec((C, H), lambda i: (0, 0)),
            pl.BlockSpec((1, C), lambda i: (0, 0)),
        ],
        out_specs=pl.BlockSpec((BLK, C), lambda i: (i, 0)),
        out_shape=jax.ShapeDtypeStruct((N, C), jnp.float32),
    )(part1, y1, s1, d1, Wc11, bc11, W3, b3)


def kernel(x, node_types, edge_index0, edge_index1, Win, b_in,
           Wc0, bc0, Wc10, bc10, Wc11, bc11, W1, b1, W2, b2, W3, b3):
    del W1, b1, W2, b2
    nt2d = node_types.astype(jnp.int32).reshape(N, 1)
    src0 = edge_index0[0].astype(jnp.int32)
    dst0 = edge_index0[1].astype(jnp.int32)
    src1 = edge_index1[0].astype(jnp.int32)
    dst1 = edge_index1[1].astype(jnp.int32)

    h0t, h1t = _sc_hist(dst0, dst1)
    y0, d0, d1 = _tc1(x, nt2d, h0t, h1t, Win, b_in)
    s0 = _sc_seg(y0, src0, dst0)
    part1, y1 = _tc2(y0, s0, d0, d1, Wc0, bc0.reshape(1, H),
                     Wc10, bc10.reshape(1, H))
    s1 = _sc_seg(y1, src1, dst1)
    return _tc3(part1, y1, s1, d1, Wc11, bc11.reshape(1, H),
                W3, b3.reshape(1, C))


# speedup vs baseline: 20.3844x; 1.2415x over previous
import functools
import math

import jax
import jax.numpy as jnp
from jax import lax
from jax.experimental import pallas as pl
from jax.experimental.pallas import tpu as pltpu
from jax.experimental.pallas import tpu_sc as plsc

N = 10000
E = 320000
H = 128
C = 16

NC = 2
NS = 16
NW = NC * NS
EPW = E // NW
EB = 96
EPWP = 10080
NBATCH = EPWP // EB
NP = 10240
RPS = NP // NS
HW = 16
PAD_DST = NP - 8

_mesh = plsc.VectorSubcoreMesh(core_axis_name="c", subcore_axis_name="s")


@functools.partial(
    pl.kernel,
    out_type=(
        jax.ShapeDtypeStruct((NC, NP, HW), jnp.float32),
        jax.ShapeDtypeStruct((NC, NP, HW), jnp.float32),
    ),
    mesh=_mesh,
    compiler_params=pltpu.CompilerParams(use_tc_tiling_on_sc=False),
    scratch_types=[
        pltpu.VMEM((NBATCH, EB), jnp.int32),
        pltpu.VMEM((NBATCH, EB), jnp.int32),
        pltpu.VMEM((EB, HW), jnp.float32),
        pltpu.VMEM((RPS, HW), jnp.float32),
        pltpu.VMEM_SHARED((NP, HW), jnp.float32),
        pltpu.VMEM_SHARED((NP, HW), jnp.float32),
        pltpu.SemaphoreType.DMA,
    ],
)
def _sc_hist(dst0_hbm, dst1_hbm, out0_hbm, out1_hbm,
             i0_v, i1_v, ones_v, zb_v, t0_sh, t1_sh, sem):
    c = lax.axis_index("c")
    s = lax.axis_index("s")
    wid = c * NS + s
    lag = 8

    def fill(i, _):
        ones_v[i, :] = jnp.ones((HW,), jnp.float32)
        return _
    lax.fori_loop(0, EB, fill, None)

    def zero(i, _):
        zb_v[i, :] = jnp.zeros((HW,), jnp.float32)
        return _
    lax.fori_loop(0, RPS, zero, None)

    pltpu.sync_copy(dst0_hbm.at[wid], i0_v)
    pltpu.sync_copy(dst1_hbm.at[wid], i1_v)

    row0 = s * RPS
    pltpu.sync_copy(zb_v, t0_sh.at[pl.ds(row0, RPS), :])
    pltpu.sync_copy(zb_v, t1_sh.at[pl.ds(row0, RPS), :])
    plsc.subcore_barrier()

    def batch(j, _):
        @pl.when(j >= lag)
        def _drain():
            pltpu.make_async_copy(ones_v, t0_sh.at[i0_v.at[0]], sem).wait()
            pltpu.make_async_copy(ones_v, t1_sh.at[i1_v.at[0]], sem).wait()
        pltpu.async_copy(ones_v, t0_sh.at[i0_v.at[j]], sem, add=True)
        pltpu.async_copy(ones_v, t1_sh.at[i1_v.at[j]], sem, add=True)
        return _
    lax.fori_loop(0, NBATCH, batch, None)
    for _ in range(lag):
        pltpu.make_async_copy(ones_v, t0_sh.at[i0_v.at[0]], sem).wait()
        pltpu.make_async_copy(ones_v, t1_sh.at[i1_v.at[0]], sem).wait()

    plsc.subcore_barrier()
    pltpu.sync_copy(t0_sh.at[pl.ds(row0, RPS), :],
                    out0_hbm.at[c, pl.ds(row0, RPS), :])
    pltpu.sync_copy(t1_sh.at[pl.ds(row0, RPS), :],
                    out1_hbm.at[c, pl.ds(row0, RPS), :])


@functools.partial(
    pl.kernel,
    out_type=jax.ShapeDtypeStruct((NC, NP, H), jnp.float32),
    mesh=_mesh,
    compiler_params=pltpu.CompilerParams(use_tc_tiling_on_sc=False),
    scratch_types=[
        pltpu.VMEM((NBATCH, EB), jnp.int32),
        pltpu.VMEM((NBATCH, EB), jnp.int32),
        pltpu.VMEM((2, EB, H), jnp.float32),
        pltpu.VMEM_SHARED((NP, H), jnp.float32),
        pltpu.SemaphoreType.DMA((2,)),
    ],
)
def _sc_seg(y_hbm, src_hbm, dst_hbm, out_hbm,
            src_v, dst_v, rows_v, table_sh, gsem):
    c = lax.axis_index("c")
    s = lax.axis_index("s")
    wid = c * NS + s

    def zero(i, _):
        for k in range(H // 16):
            rows_v[0, i, pl.ds(k * 16, 16)] = jnp.zeros((16,), jnp.float32)
        return _
    lax.fori_loop(0, EB, zero, None)

    pltpu.sync_copy(src_hbm.at[wid], src_v)
    pltpu.sync_copy(dst_hbm.at[wid], dst_v)

    row0 = s * RPS
    for r in range(RPS // EB):
        pltpu.sync_copy(rows_v.at[0],
                        table_sh.at[pl.ds(row0 + r * EB, EB), :])
    pltpu.sync_copy(rows_v.at[0, pl.ds(0, RPS - (RPS // EB) * EB)],
                    table_sh.at[pl.ds(row0 + (RPS // EB) * EB,
                                      RPS - (RPS // EB) * EB), :])
    plsc.subcore_barrier()

    def batch(j, _):
        pltpu.async_copy(y_hbm.at[src_v.at[j]], rows_v.at[0],
                         gsem.at[0]).wait()
        pltpu.sync_copy(rows_v.at[0], table_sh.at[dst_v.at[j]], add=True)
        return _
    lax.fori_loop(0, NBATCH, batch, None)

    plsc.subcore_barrier()
    pltpu.sync_copy(table_sh.at[pl.ds(row0, RPS), :],
                    out_hbm.at[c, pl.ds(row0, RPS), :])


BLK = 1000
GRID = N // BLK


def _tc1_body(x_ref, nt_ref, h0_ref, h1_ref, win_ref, bin_ref,
              y0_ref, d0_ref, d1_ref):
    x = x_ref[...]
    nt = nt_ref[...]
    hid = jnp.zeros_like(x)
    for t in range(4):
        p = lax.dot_general(x, win_ref[t], (((1,), (1,)), ((), ())),
                            preferred_element_type=jnp.float32)
        p = p + bin_ref[t][None, :]
        hid = jnp.where(nt == t, p, hid)
    deg0 = h0_ref[0, :, 0:1] + h0_ref[1, :, 0:1] + 1.0
    deg1 = h1_ref[0, :, 0:1] + h1_ref[1, :, 0:1] + 1.0
    d0 = lax.rsqrt(deg0)
    d1 = lax.rsqrt(deg1)
    y0_ref[...] = hid * d0
    d0_ref[...] = d0
    d1_ref[...] = d1


def _tc1(x, nt2d, h0t, h1t, Win, b_in):
    return pl.pallas_call(
        _tc1_body,
        grid=(GRID,),
        in_specs=[
            pl.BlockSpec((BLK, H), lambda i: (i, 0)),
            pl.BlockSpec((BLK, 1), lambda i: (i, 0)),
            pl.BlockSpec((NC, BLK, HW), lambda i: (0, i, 0)),
            pl.BlockSpec((NC, BLK, HW), lambda i: (0, i, 0)),
            pl.BlockSpec((4, H, H), lambda i: (0, 0, 0)),
            pl.BlockSpec((4, H), lambda i: (0, 0)),
        ],
        out_specs=[
            pl.BlockSpec((BLK, H), lambda i: (i, 0)),
            pl.BlockSpec((BLK, 1), lambda i: (i, 0)),
            pl.BlockSpec((BLK, 1), lambda i: (i, 0)),
        ],
        out_shape=[
            jax.ShapeDtypeStruct((N, H), jnp.float32),
            jax.ShapeDtypeStruct((N, 1), jnp.float32),
            jax.ShapeDtypeStruct((N, 1), jnp.float32),
        ],
    )(x, nt2d, h0t, h1t, Win, b_in)


def _tc2_body(y0_ref, s0_ref, d0_ref, d1_ref, wc0_ref, bc0_ref,
              wc10_ref, bc10_ref, part1_ref, y1_ref):
    agg0 = d0_ref[...] * (s0_ref[0] + s0_ref[1] + y0_ref[...])
    h0 = lax.dot_general(agg0, wc0_ref[...], (((1,), (1,)), ((), ())),
                         preferred_element_type=jnp.float32) + bc0_ref[...]
    part1 = lax.dot_general(agg0, wc10_ref[...], (((1,), (1,)), ((), ())),
                            preferred_element_type=jnp.float32) + bc10_ref[...]
    part1_ref[...] = part1
    y1_ref[...] = d1_ref[...] * h0


def _tc2(y0, s0, d0, d1, Wc0, bc0, Wc10, bc10):
    return pl.pallas_call(
        _tc2_body,
        grid=(GRID,),
        in_specs=[
            pl.BlockSpec((BLK, H), lambda i: (i, 0)),
            pl.BlockSpec((NC, BLK, H), lambda i: (0, i, 0)),
            pl.BlockSpec((BLK, 1), lambda i: (i, 0)),
            pl.BlockSpec((BLK, 1), lambda i: (i, 0)),
            pl.BlockSpec((H, H), lambda i: (0, 0)),
            pl.BlockSpec((1, H), lambda i: (0, 0)),
            pl.BlockSpec((H, H), lambda i: (0, 0)),
            pl.BlockSpec((1, H), lambda i: (0, 0)),
        ],
        out_specs=[
            pl.BlockSpec((BLK, H), lambda i: (i, 0)),
            pl.BlockSpec((BLK, H), lambda i: (i, 0)),
        ],
        out_shape=[
            jax.ShapeDtypeStruct((N, H), jnp.float32),
            jax.ShapeDtypeStruct((N, H), jnp.float32),
        ],
    )(y0, s0, d0, d1, Wc0, bc0, Wc10, bc10)


def _tc3_body(part1_ref, y1_ref, s1_ref, d1_ref, wc11_ref, bc11_ref,
              w3_ref, b3_ref, out_ref):
    agg1 = d1_ref[...] * (s1_ref[0] + s1_ref[1] + y1_ref[...])
    h1 = part1_ref[...] + lax.dot_general(
        agg1, wc11_ref[...], (((1,), (1,)), ((), ())),
        preferred_element_type=jnp.float32) + bc11_ref[...]
    mu = jnp.mean(h1, axis=-1, keepdims=True)
    cent = h1 - mu
    var = jnp.mean(cent * cent, axis=-1, keepdims=True)
    tnorm = cent * lax.rsqrt(var + 1e-5)
    g = 0.5 * tnorm * (1.0 + lax.erf(tnorm * (1.0 / math.sqrt(2.0))))
    out_ref[...] = lax.dot_general(
        g, w3_ref[...], (((1,), (1,)), ((), ())),
        preferred_element_type=jnp.float32) + b3_ref[...]


def _tc3(part1, y1, s1, d1, Wc11, bc11, W3, b3):
    return pl.pallas_call(
        _tc3_body,
        grid=(GRID,),
        in_specs=[
            pl.BlockSpec((BLK, H), lambda i: (i, 0)),
            pl.BlockSpec((BLK, H), lambda i: (i, 0)),
            pl.BlockSpec((NC, BLK, H), lambda i: (0, i, 0)),
            pl.BlockSpec((BLK, 1), lambda i: (i, 0)),
            pl.BlockSpec((H, H), lambda i: (0, 0)),
            pl.BlockSpec((1, H), lambda i: (0, 0)),
            pl.BlockSpec((C, H), lambda i: (0, 0)),
            pl.BlockSpec((1, C), lambda i: (0, 0)),
        ],
        out_specs=pl.BlockSpec((BLK, C), lambda i: (i, 0)),
        out_shape=jax.ShapeDtypeStruct((N, C), jnp.float32),
    )(part1, y1, s1, d1, Wc11, bc11, W3, b3)


def kernel(x, node_types, edge_index0, edge_index1, Win, b_in,
           Wc0, bc0, Wc10, bc10, Wc11, bc11, W1, b1, W2, b2, W3, b3):
    del W1, b1, W2, b2
    nt2d = node_types.astype(jnp.int32).reshape(N, 1)

    def pad3(a, fill):
        a2 = a.astype(jnp.int32).reshape(NW, EPW)
        a2 = jnp.pad(a2, ((0, 0), (0, EPWP - EPW)), constant_values=fill)
        return a2.reshape(NW, NBATCH, EB)

    src0 = pad3(edge_index0[0], 0)
    dst0 = pad3(edge_index0[1], PAD_DST)
    src1 = pad3(edge_index1[0], 0)
    dst1 = pad3(edge_index1[1], PAD_DST)

    h0t, h1t = _sc_hist(dst0, dst1)
    y0, d0, d1 = _tc1(x, nt2d, h0t, h1t, Win, b_in)
    s0 = _sc_seg(y0, src0, dst0)
    part1, y1 = _tc2(y0, s0, d0, d1, Wc0, bc0.reshape(1, H),
                     Wc10, bc10.reshape(1, H))
    s1 = _sc_seg(y1, src1, dst1)
    return _tc3(part1, y1, s1, d1, Wc11, bc11.reshape(1, H),
                W3, b3.reshape(1, C))
